# baseline (speedup 1.0000x reference)
import jax
import jax.numpy as jnp
from jax import lax
from jax.experimental import pallas as pl
from jax.experimental.pallas import tpu as pltpu
from jax.experimental.pallas import tpu_sc as plsc

N = 50000
E = 800000
D = 64
H = 32
NP = 50176
EC = E // 16
K = 512
B = 2048
NBLK = EC // B
REM = EC - NBLK * B
TAIL = REM - K
PAD_CNT = float(16 * (K - TAIL))
ROWS_PER_TILE = NP // 16
ZCOL = 392


def _proj_body(x_ref, bd0_ref, bd1_ref, y_ref):
    x = x_ref[...]
    y_ref[0] = jnp.dot(x, bd0_ref[...], preferred_element_type=jnp.float32)
    y_ref[1] = jnp.dot(x, bd1_ref[...], preferred_element_type=jnp.float32)


def _project(xp, bd0, bd1):
    return pl.pallas_call(
        _proj_body,
        grid=(NP // 1024,),
        in_specs=[
            pl.BlockSpec((256, 4 * D), lambda i: (i, 0)),
            pl.BlockSpec((4 * D, 128), lambda i: (0, 0)),
            pl.BlockSpec((4 * D, 128), lambda i: (0, 0)),
        ],
        out_specs=pl.BlockSpec((2, 256, 128), lambda i: (0, i, 0)),
        out_shape=jax.ShapeDtypeStruct((2, NP * H // 128, 128), jnp.float32),
    )(xp, bd0, bd1)


def _sc_body(y_hbm, edge_hbm, b1_hbm, s_hbm,
             acc, histsp, srcbuf, dstbuf, msg, ones, degbuf, b1buf,
             semg, sems):
    c = lax.axis_index("c")
    s = lax.axis_index("s")
    z16 = jnp.zeros((16,), jnp.float32)
    one16 = jnp.full((16,), 1.0, jnp.float32)

    for k in range(8):
        ones[pl.ds(k * 16, 16)] = one16

    def zero_deg(r, _):
        degbuf[pl.ds(r * 16, 16)] = z16
        return _
    lax.fori_loop(0, ROWS_PER_TILE // 16, zero_deg, None)

    def zero_msg(r, _):
        msg[r, pl.ds(0, 16)] = z16
        msg[r, pl.ds(16, 16)] = z16
        return _
    lax.fori_loop(0, K, zero_msg, None)

    for k in range(6):
        pltpu.sync_copy(msg, acc.at[pl.ds(s * ROWS_PER_TILE + k * K, K)])
    pltpu.sync_copy(msg.at[pl.ds(0, ROWS_PER_TILE - 6 * K)],
                    acc.at[pl.ds(s * ROWS_PER_TILE + 6 * K,
                                 ROWS_PER_TILE - 6 * K)])
    pltpu.sync_copy(degbuf, histsp.at[pl.ds(s * ROWS_PER_TILE, ROWS_PER_TILE)])
    plsc.subcore_barrier()

    ebase = s * EC
    CK = 256
    UPB = B // CK

    def issue_gathers(u):
        slot = u % 2
        return [pltpu.async_copy(
                    y_hbm.at[c].at[srcbuf.at[pl.ds(u * CK + j * 128, 128)]],
                    msg.at[pl.ds(slot * CK + j * 128, 128)], semg[slot])
                for j in range(2)]

    def issue_scatters(u):
        slot = u % 2
        return [pltpu.async_copy(
                    msg.at[pl.ds(slot * CK + j * 128, 128)],
                    acc.at[dstbuf.at[pl.ds(u * CK + j * 128, 128)]],
                    sems[slot], add=True)
                for j in range(2)]

    def do_hist(u):
        for j in range(2):
            pltpu.sync_copy(
                ones, histsp.at[srcbuf.at[pl.ds(u * CK + j * 128, 128)]],
                add=True)

    def block(b, _):
        off = ebase + b * B
        pltpu.sync_copy(edge_hbm.at[0, pl.ds(off, B)], srcbuf)
        pltpu.sync_copy(edge_hbm.at[1, pl.ds(off, B)], dstbuf)
        g_cps = issue_gathers(0)
        s_cps = [None, None]
        for u in range(UPB):
            if u + 1 < UPB:
                if s_cps[(u + 1) % 2] is not None:
                    for cp in s_cps[(u + 1) % 2]:
                        cp.wait()
                ng = issue_gathers(u + 1)
            for cp in g_cps:
                cp.wait()
            s_cps[u % 2] = issue_scatters(u)
            do_hist(u)
            if u + 1 < UPB:
                g_cps = ng
        for slot in range(2):
            for cp in s_cps[slot]:
                cp.wait()
        return _
    lax.fori_loop(0, NBLK, block, None)

    def fill_tail(r, _):
        srcbuf[pl.ds(REM + r * 16, 16)] = jnp.zeros((16,), jnp.int32)
        dstbuf[pl.ds(REM + r * 16, 16)] = jnp.full((16,), N, jnp.int32)
        return _
    lax.fori_loop(0, (4 * CK - REM) // 16, fill_tail, None)
    roff = ebase + NBLK * B
    pltpu.sync_copy(edge_hbm.at[0, pl.ds(roff, REM)], srcbuf.at[pl.ds(0, REM)])
    pltpu.sync_copy(edge_hbm.at[1, pl.ds(roff, REM)], dstbuf.at[pl.ds(0, REM)])
    for u in range(4):
        for cp in issue_gathers(u):
            cp.wait()
        for cp in issue_scatters(u):
            cp.wait()
        do_hist(u)

    plsc.subcore_barrier()

    pltpu.sync_copy(histsp.at[pl.ds(s * ROWS_PER_TILE, ROWS_PER_TILE)], degbuf)
    pltpu.sync_copy(b1_hbm.at[c], b1buf)

    @pl.when(s == 0)
    def _():
        v = degbuf[pl.ds(0, 16)]
        lane = lax.iota(jnp.int32, 16)
        degbuf[pl.ds(0, 16)] = v - jnp.where(lane == 0, PAD_CNT, 0.0)

    b1lo = b1buf[pl.ds(0, 16)]
    b1hi = b1buf[pl.ds(16, 16)]
    nrows = jnp.minimum(ROWS_PER_TILE, N - s * ROWS_PER_TILE)

    def weigh_rows(carry_chunk):
        k, nch = carry_chunk
        rcount = jnp.clip(nrows - k * 512, 0, nch)

        def row(r, sacc):
            a0, a1 = sacc
            dj = plsc.load_gather(degbuf, [jnp.full((16,), k * 512 + r,
                                                    jnp.int32)])
            m0 = msg[r, pl.ds(0, 16)]
            m1 = msg[r, pl.ds(16, 16)]
            a0 = a0 + dj * jnp.maximum(m0 + b1lo, 0.0)
            a1 = a1 + dj * jnp.maximum(m1 + b1hi, 0.0)
            return (a0, a1)
        return rcount, row

    acc0 = jnp.zeros((16,), jnp.float32)
    acc1 = jnp.zeros((16,), jnp.float32)
    for k in range(7):
        nch = 512 if k < 6 else 64
        pltpu.sync_copy(acc.at[pl.ds(s * ROWS_PER_TILE + k * 512, nch)],
                        msg.at[pl.ds(0, nch)])
        rcount, row = weigh_rows((k, nch))
        acc0, acc1 = lax.fori_loop(0, rcount, row, (acc0, acc1))

    ones[pl.ds(0, 16)] = acc0
    ones[pl.ds(16, 16)] = acc1
    pltpu.sync_copy(ones.at[pl.ds(0, 32)], s_hbm.at[c, s])


def _sc_aggregate(y, edge_index, b1):
    mesh = plsc.VectorSubcoreMesh(core_axis_name="c", subcore_axis_name="s")
    fn = pl.kernel(
        _sc_body,
        out_type=jax.ShapeDtypeStruct((2, 16, H), jnp.float32),
        mesh=mesh,
        compiler_params=pltpu.CompilerParams(
            needs_layout_passes=False, use_tc_tiling_on_sc=False),
        scratch_types=[
            pltpu.VMEM_SHARED((NP, H), jnp.float32),
            pltpu.VMEM_SHARED((NP,), jnp.float32),
            pltpu.VMEM((B,), jnp.int32),
            pltpu.VMEM((B,), jnp.int32),
            pltpu.VMEM((K, H), jnp.float32),
            pltpu.VMEM((128,), jnp.float32),
            pltpu.VMEM((ROWS_PER_TILE,), jnp.float32),
            pltpu.VMEM((H,), jnp.float32),
            [pltpu.SemaphoreType.DMA] * 2,
            [pltpu.SemaphoreType.DMA] * 2,
        ],
    )
    return fn(y, edge_index, b1)


def _finish_body(s_ref, w2a_ref, w2b_ref, b2_ref, out_ref):
    s0 = jnp.sum(s_ref[0], axis=0, keepdims=True)
    s1 = jnp.sum(s_ref[1], axis=0, keepdims=True)
    out = (jnp.dot(s0, w2a_ref[...], preferred_element_type=jnp.float32)
           + jnp.dot(s1, w2b_ref[...], preferred_element_type=jnp.float32))
    out_ref[...] = out * (1.0 / N) + b2_ref[...]


def _finish(sp, w2a, w2b, b2):
    return pl.pallas_call(
        _finish_body,
        out_shape=jax.ShapeDtypeStruct((1, D), jnp.float32),
    )(sp, w2a, w2b, b2)


@jax.jit
def kernel(feats, edge_index, W1, b1, W2, b2):
    xp = feats.reshape(N // 4, 4 * D)
    z = jnp.zeros((4 * D, 128), jnp.float32)
    bd0 = z
    bd1 = z
    for k in range(4):
        bd0 = bd0.at[k * D:(k + 1) * D, k * H:(k + 1) * H].set(W1[:, :H])
        bd1 = bd1.at[k * D:(k + 1) * D, k * H:(k + 1) * H].set(W1[:, H:])
    y = _project(xp, bd0, bd1).reshape(2, NP, H)
    sp = _sc_aggregate(y, edge_index, b1.reshape(2, H))
    return _finish(sp, W2[:H, :], W2[H:, :], b2.reshape(1, D))
